# Initial kernel scaffold; baseline (speedup 1.0000x reference)
import jax
import jax.numpy as jnp
from jax.experimental import pallas as pl


def kernel(x, y, u_ew, d_ew, nearest_nodes, mu_u, mu_d2, rho, alpha_x, beta_x):
    raise NotImplementedError("write your pallas kernel here")


# speedup vs baseline: 26.9365x; 26.9365x over previous
import jax
import jax.numpy as jnp
from jax import lax
from jax.experimental import pallas as pl
from jax.experimental.pallas import tpu as pltpu, tpu_sc as plsc

T = 12
T_OBS = 6
N = 10000
H = 2
C = 4
K1 = 17
NPAD = 10240
NT = 16
NPT = NPAD // NT
NG = NPT // 16


def _sc_body(xT, rhsT, uT, dT, nnT, params, x0, r, pA, pB,
             slab, nn_own, u_own, d_own, lu_t, zc_t, zp_t, if_t, tmpA, pown,
             slots, prm):
    acc = slab
    h = lax.axis_index("c")
    sid = lax.axis_index("s")
    own = pl.ds(sid * NPT, NPT)
    csplat = [jnp.full((16,), c, jnp.int32) for c in range(C)]

    pltpu.sync_copy(nnT.at[:, own], nn_own)
    pltpu.sync_copy(params, prm)
    mu = prm[h, 0]
    c2 = prm[h, 1]

    def vloop(ncols, body):
        for c in range(C):
            def vb(j, _):
                body(c, pl.ds(j * 16, 16))
                return 0
            lax.fori_loop(0, ncols // 16, vb, 0)

    def zero2d(ref, ncols):
        def zb(c, sl):
            ref[c, sl] = jnp.zeros((16,), jnp.float32)
        vloop(ncols, zb)

    zero2d(zp_t, NPT)
    zero2d(if_t, NPT)

    def gather_groups(has_z):
        def grp(g, _):
            off = g * 16
            gsl = pl.ds(off, 16)

            def kstep(k, carry):
                au0, au1, au2, au3, ad0, ad1, ad2, ad3 = carry
                idx = nn_own[k, gsl]
                wu = u_own[k, gsl]
                wd = d_own[k, gsl]
                g0 = plsc.load_gather(slab, [csplat[0], idx])
                g1 = plsc.load_gather(slab, [csplat[1], idx])
                g2 = plsc.load_gather(slab, [csplat[2], idx])
                g3 = plsc.load_gather(slab, [csplat[3], idx])
                return (au0 + wu * g0, au1 + wu * g1,
                        au2 + wu * g2, au3 + wu * g3,
                        ad0 + wd * g0, ad1 + wd * g1,
                        ad2 + wd * g2, ad3 + wd * g3)

            zv = jnp.zeros((16,), jnp.float32)
            au0, au1, au2, au3, ad0, ad1, ad2, ad3 = lax.fori_loop(
                0, K1, kstep, (zv,) * 8)
            aus = (au0, au1, au2, au3)
            ads = (ad0, ad1, ad2, ad3)
            for c in range(C):
                pv = slab[c, pl.ds(sid * NPT + off, 16)]
                pown[c, gsl] = pv
                lu_t[c, gsl] = pv - aus[c]
            @pl.when(has_z)
            def _():
                for c in range(C):
                    zc_t[c, gsl] = zc_t[c, gsl] - ads[c]
            return 0

        lax.fori_loop(0, NG, grp, 0)

    def scatter_groups():
        def grp(g, _):
            gsl = pl.ds(g * 16, 16)
            z0 = zc_t[0, gsl]
            z1 = zc_t[1, gsl]
            z2 = zc_t[2, gsl]
            z3 = zc_t[3, gsl]

            def kstep(k, _):
                idx = nn_own[k, gsl]
                wd = d_own[k, gsl]
                plsc.addupdate_scatter(acc, [csplat[0], idx], wd * z0)
                plsc.addupdate_scatter(acc, [csplat[1], idx], wd * z1)
                plsc.addupdate_scatter(acc, [csplat[2], idx], wd * z2)
                plsc.addupdate_scatter(acc, [csplat[3], idx], wd * z3)
                return 0

            lax.fori_loop(0, K1, kstep, 0)
            return 0

        lax.fori_loop(0, NG, grp, 0)

    def reduce_exchange():
        pltpu.sync_copy(acc, slots.at[sid])
        plsc.subcore_barrier()
        zero2d(if_t, NPT)

        def red(j, _):
            pltpu.sync_copy(slots.at[j, :, own], tmpA)

            def addv(c, sl):
                if_t[c, sl] = if_t[c, sl] + tmpA[c, sl]
            vloop(NPT, addv)
            return 0

        lax.fori_loop(0, NT, red, 0)
        plsc.subcore_barrier()

    def ap_vec(t, c, j):
        sl = pl.ds(j * 16, 16)
        pv = pown[c, sl]
        lu = lu_t[c, sl]
        zp = zp_t[c, sl]
        inf = if_t[c, sl]
        hth = jnp.where(t < T_OBS, 1.0, 0.0).astype(jnp.float32)
        wz = jnp.where(t >= 1, 1.0, 0.0).astype(jnp.float32)
        wi = jnp.where(t <= T - 2, 1.0, 0.0).astype(jnp.float32)
        cldr = wz * zp - wi * inf
        return hth * pv + mu * lu + c2 * cldr, pv

    def epilogue_A(t):
        pltpu.sync_copy(rhsT.at[t, h, :, own], tmpA)

        for c in range(C):
            def upd(j, _, c=c):
                ap, _pv = ap_vec(t, c, j)
                sl = pl.ds(j * 16, 16)
                tmpA[c, sl] = tmpA[c, sl] - ap
                return 0
            lax.fori_loop(0, NG, upd, 0)
        pltpu.sync_copy(tmpA, r.at[t, h, :, own])
        pltpu.sync_copy(tmpA, pA.at[t, h, :, own])
        pltpu.sync_copy(xT.at[t, h, :, own], x0.at[t, h, :, own])

    def make_epilogue_B(i, pdst):
        a = prm[h, 2 + i]
        b = prm[h, 5 + i]

        def epilogue(t):
            pltpu.sync_copy(x0.at[t, h, :, own], tmpA)

            for c in range(C):
                def updx(j, _, c=c):
                    _ap, pv = ap_vec(t, c, j)
                    sl = pl.ds(j * 16, 16)
                    tmpA[c, sl] = tmpA[c, sl] + a * pv
                    return 0
                lax.fori_loop(0, NG, updx, 0)
            pltpu.sync_copy(tmpA, x0.at[t, h, :, own])

            pltpu.sync_copy(r.at[t, h, :, own], tmpA)
            for c in range(C):
                def updr(j, _, c=c):
                    ap, pv = ap_vec(t, c, j)
                    sl = pl.ds(j * 16, 16)
                    rn = tmpA[c, sl] - a * ap
                    tmpA[c, sl] = rn
                    lu_t[c, sl] = rn + b * pv
                    return 0
                lax.fori_loop(0, NG, updr, 0)
            pltpu.sync_copy(tmpA, r.at[t, h, :, own])
            pltpu.sync_copy(lu_t, pdst.at[t, h, :, own])
        return epilogue

    def lhs_frame(t, src, epilogue):
        pltpu.sync_copy(src.at[t, h], slab)
        pltpu.sync_copy(uT.at[t, h, :, own], u_own)
        has_z = t <= T - 2

        @pl.when(has_z)
        def _():
            pltpu.sync_copy(dT.at[t, h, :, own], d_own)
            pltpu.sync_copy(src.at[t + 1, h, :, own], zc_t)
        gather_groups(has_z)

        @pl.when(has_z)
        def _():
            zero2d(acc, NPAD)
            scatter_groups()
            reduce_exchange()
        epilogue(t)

        @pl.when(has_z)
        def _():
            def cpv(c, sl):
                zp_t[c, sl] = zc_t[c, sl]
            vloop(NPT, cpv)

    plsc.subcore_barrier()

    def run_pass(src, epilogue):
        def frame(t, _):
            lhs_frame(t, src, epilogue)
            return 0
        lax.fori_loop(0, T, frame, 0)
        plsc.subcore_barrier()

    run_pass(xT, epilogue_A)
    run_pass(pA, make_epilogue_B(0, pB))
    run_pass(pB, make_epilogue_B(1, pA))
    run_pass(pA, make_epilogue_B(2, pB))


@jax.jit
def kernel(x, y, u_ew, d_ew, nearest_nodes, mu_u, mu_d2, rho, alpha_x, beta_x):
    f32 = jnp.float32

    def pad_nodes(a):
        pw = [(0, 0)] * (a.ndim - 1) + [(0, NPAD - N)]
        return jnp.pad(a, pw)

    xT = pad_nodes(jnp.transpose(x[0], (0, 2, 3, 1)))
    rhsT = jnp.concatenate(
        [pad_nodes(jnp.transpose(y[0], (0, 2, 3, 1))),
         jnp.zeros((T - T_OBS, H, C, NPAD), f32)], axis=0)
    uTr = jnp.transpose(u_ew[0], (0, 3, 2, 1))
    uT = pad_nodes(jnp.concatenate(
        [jnp.zeros((T, H, 1, N), f32), uTr], axis=2))
    dT = pad_nodes(jnp.transpose(d_ew[0], (0, 3, 2, 1)))
    nnT = jnp.pad(nearest_nodes.T, ((0, 0), (0, NPAD - N)))

    a_h = alpha_x[0, :, :, 0].T
    b_h = beta_x[0, :, :, 0].T
    params = jnp.concatenate(
        [jnp.full((H, 1), mu_u[0], f32),
         jnp.full((H, 1), mu_d2[0] + rho[0] / 2.0, f32),
         a_h.astype(f32), b_h.astype(f32)], axis=1)
    params = jnp.broadcast_to(params[:, :, None], (H, 8, 16))

    mesh = plsc.VectorSubcoreMesh(core_axis_name="c", subcore_axis_name="s")
    out = pl.kernel(
        _sc_body,
        out_type=(
            jax.ShapeDtypeStruct((T, H, C, NPAD), f32),
            jax.ShapeDtypeStruct((T, H, C, NPAD), f32),
            jax.ShapeDtypeStruct((T, H, C, NPAD), f32),
            jax.ShapeDtypeStruct((T, H, C, NPAD), f32),
        ),
        mesh=mesh,
        compiler_params=pltpu.CompilerParams(use_tc_tiling_on_sc=False,
                                             needs_layout_passes=False),
        scratch_types=[
            pltpu.VMEM((C, NPAD), f32),
            pltpu.VMEM((K1, NPT), jnp.int32),
            pltpu.VMEM((K1, NPT), f32),
            pltpu.VMEM((K1, NPT), f32),
            pltpu.VMEM((C, NPT), f32),
            pltpu.VMEM((C, NPT), f32),
            pltpu.VMEM((C, NPT), f32),
            pltpu.VMEM((C, NPT), f32),
            pltpu.VMEM((C, NPT), f32),
            pltpu.VMEM((C, NPT), f32),
            pltpu.VMEM_SHARED((NT, C, NPAD), f32),
            pltpu.VMEM((H, 8, 16), f32),
        ],
    )(xT, rhsT, uT, dT, nnT, params)

    x0 = out[0][:, :, :, :N]
    return jnp.transpose(x0, (0, 3, 1, 2))[None]


# speedup vs baseline: 57.4690x; 2.1335x over previous
import jax
import jax.numpy as jnp
from jax import lax
from jax.experimental import pallas as pl
from jax.experimental.pallas import tpu as pltpu, tpu_sc as plsc

T = 12
T_OBS = 6
N = 10000
H = 2
C = 4
K1 = 17
NPAD = 10240
NT = 16
NPT = NPAD // NT
NG = NPT // 16


def _sc_body(xT, rhsT, uT, dT, nnT, params, x0, r, pA, pB, slots,
             slab, nn_own, u_own, d_own, lu_t, zc_t, zp_t, if_t, tmpA, pown,
             rbuf, prm, sem):
    acc = slab
    h = lax.axis_index("c")
    sid = lax.axis_index("s")
    own = pl.ds(sid * NPT, NPT)
    csplat = [jnp.full((16,), c, jnp.int32) for c in range(C)]

    pltpu.sync_copy(nnT.at[:, own], nn_own)
    pltpu.sync_copy(params, prm)
    mu = prm[h, 0]
    c2 = prm[h, 1]

    def vloop(ncols, body, unroll=8):
        for c in range(C):
            @plsc.parallel_loop(0, ncols // 16, unroll=unroll)
            def _vb(j, c=c):
                body(c, pl.ds(j * 16, 16))

    def zero2d(ref, ncols):
        def zb(c, sl):
            ref[c, sl] = jnp.zeros((16,), jnp.float32)
        vloop(ncols, zb)

    zero2d(zp_t, NPT)
    zero2d(if_t, NPT)

    def gather_groups(has_z):
        def grp(g, _):
            off = g * 16
            gsl = pl.ds(off, 16)
            zv = jnp.zeros((16,), jnp.float32)

            @plsc.parallel_loop(0, K1, carry=(zv,) * 8)
            def kstep(k, carry):
                au0, au1, au2, au3, ad0, ad1, ad2, ad3 = carry
                idx = nn_own[k, gsl]
                wu = u_own[k, gsl]
                wd = d_own[k, gsl]
                g0 = plsc.load_gather(slab, [csplat[0], idx])
                g1 = plsc.load_gather(slab, [csplat[1], idx])
                g2 = plsc.load_gather(slab, [csplat[2], idx])
                g3 = plsc.load_gather(slab, [csplat[3], idx])
                return (au0 + wu * g0, au1 + wu * g1,
                        au2 + wu * g2, au3 + wu * g3,
                        ad0 + wd * g0, ad1 + wd * g1,
                        ad2 + wd * g2, ad3 + wd * g3)

            au0, au1, au2, au3, ad0, ad1, ad2, ad3 = kstep
            aus = (au0, au1, au2, au3)
            ads = (ad0, ad1, ad2, ad3)
            for c in range(C):
                pv = slab[c, pl.ds(sid * NPT + off, 16)]
                pown[c, gsl] = pv
                lu_t[c, gsl] = pv - aus[c]
            @pl.when(has_z)
            def _():
                for c in range(C):
                    zc_t[c, gsl] = zc_t[c, gsl] - ads[c]
            return 0

        lax.fori_loop(0, NG, grp, 0)

    def scatter_groups():
        def grp(g, _):
            gsl = pl.ds(g * 16, 16)
            z0 = zc_t[0, gsl]
            z1 = zc_t[1, gsl]
            z2 = zc_t[2, gsl]
            z3 = zc_t[3, gsl]

            @plsc.parallel_loop(0, K1)
            def kstep(k):
                idx = nn_own[k, gsl]
                wd = d_own[k, gsl]
                plsc.addupdate_scatter(acc, [csplat[0], idx], wd * z0)
                plsc.addupdate_scatter(acc, [csplat[1], idx], wd * z1)
                plsc.addupdate_scatter(acc, [csplat[2], idx], wd * z2)
                plsc.addupdate_scatter(acc, [csplat[3], idx], wd * z3)
            return 0

        lax.fori_loop(0, NG, grp, 0)

    def reduce_exchange():
        with jax.named_scope("exw"):
            wdesc = pltpu.async_copy(acc, slots.at[h, sid], sem)
            zero2d(if_t, NPT)
            wdesc.wait()
        with jax.named_scope("exb"):
            plsc.subcore_barrier()
        def red(qq, _):
            ds_ = [pltpu.async_copy(
                slots.at[h, qq * 4 + jj, :, own], rbuf.at[jj], sem)
                for jj in range(4)]
            for dd in ds_:
                dd.wait()

            def addv(c, sl):
                if_t[c, sl] = (if_t[c, sl]
                               + (rbuf[0, c, sl] + rbuf[1, c, sl])
                               + (rbuf[2, c, sl] + rbuf[3, c, sl]))
            vloop(NPT, addv)
            return 0

        with jax.named_scope("exr"):
            lax.fori_loop(0, NT // 4, red, 0)
        with jax.named_scope("exb2"):
            plsc.subcore_barrier()

    def ap_vec_sl(t, c, sl):
        pv = pown[c, sl]
        lu = lu_t[c, sl]
        zp = zp_t[c, sl]
        inf = if_t[c, sl]
        hth = jnp.where(t < T_OBS, 1.0, 0.0).astype(jnp.float32)
        wz = jnp.where(t >= 1, 1.0, 0.0).astype(jnp.float32)
        wi = jnp.where(t <= T - 2, 1.0, 0.0).astype(jnp.float32)
        cldr = wz * zp - wi * inf
        return hth * pv + mu * lu + c2 * cldr, pv

    def epilogue_A(t):
        pltpu.sync_copy(rhsT.at[t, h, :, own], tmpA)

        def upd(c, sl):
            ap, _pv = ap_vec_sl(t, c, sl)
            tmpA[c, sl] = tmpA[c, sl] - ap
        vloop(NPT, upd)
        pltpu.sync_copy(tmpA, r.at[t, h, :, own])
        pltpu.sync_copy(tmpA, pA.at[t, h, :, own])
        pltpu.sync_copy(xT.at[t, h, :, own], x0.at[t, h, :, own])

    def make_epilogue_B(i, pdst):
        a = prm[h, 2 + i]
        b = prm[h, 5 + i]

        def epilogue(t):
            e1 = pltpu.async_copy(x0.at[t, h, :, own], tmpA, sem)
            e2 = pltpu.async_copy(r.at[t, h, :, own], rbuf.at[0], sem)
            e1.wait()
            e2.wait()

            def upd(c, sl):
                ap, pv = ap_vec_sl(t, c, sl)
                tmpA[c, sl] = tmpA[c, sl] + a * pv
                rn = rbuf[0, c, sl] - a * ap
                rbuf[0, c, sl] = rn
                lu_t[c, sl] = rn + b * pv
            vloop(NPT, upd)
            pltpu.sync_copy(tmpA, x0.at[t, h, :, own])
            pltpu.sync_copy(rbuf.at[0], r.at[t, h, :, own])
            pltpu.sync_copy(lu_t, pdst.at[t, h, :, own])
        return epilogue

    def lhs_frame(t, src, epilogue):
        with jax.named_scope("stage"):
            d1 = pltpu.async_copy(src.at[t, h], slab, sem)
            d2 = pltpu.async_copy(uT.at[t, h, :, own], u_own, sem)
            has_z = t <= T - 2

            @pl.when(has_z)
            def _():
                d3 = pltpu.async_copy(dT.at[t, h, :, own], d_own, sem)
                d4 = pltpu.async_copy(src.at[t + 1, h, :, own], zc_t, sem)
                d3.wait()
                d4.wait()
            d1.wait()
            d2.wait()
        with jax.named_scope("gather"):
            gather_groups(has_z)

        @pl.when(has_z)
        def _():
            with jax.named_scope("zeroacc"):
                zero2d(acc, NPAD)
            with jax.named_scope("scatter"):
                scatter_groups()
            with jax.named_scope("exchange"):
                reduce_exchange()
        with jax.named_scope("epilogue"):
            epilogue(t)

        @pl.when(has_z)
        def _():
            def cpv(c, sl):
                zp_t[c, sl] = zc_t[c, sl]
            vloop(NPT, cpv)

    plsc.subcore_barrier()

    def run_pass(src, epilogue):
        def frame(t, _):
            lhs_frame(t, src, epilogue)
            return 0
        lax.fori_loop(0, T, frame, 0)
        plsc.subcore_barrier()

    run_pass(xT, epilogue_A)
    run_pass(pA, make_epilogue_B(0, pB))
    run_pass(pB, make_epilogue_B(1, pA))
    run_pass(pA, make_epilogue_B(2, pB))


@jax.jit
def kernel(x, y, u_ew, d_ew, nearest_nodes, mu_u, mu_d2, rho, alpha_x, beta_x):
    f32 = jnp.float32

    def pad_nodes(a):
        pw = [(0, 0)] * (a.ndim - 1) + [(0, NPAD - N)]
        return jnp.pad(a, pw)

    xT = pad_nodes(jnp.transpose(x[0], (0, 2, 3, 1)))
    rhsT = jnp.concatenate(
        [pad_nodes(jnp.transpose(y[0], (0, 2, 3, 1))),
         jnp.zeros((T - T_OBS, H, C, NPAD), f32)], axis=0)
    uTr = jnp.transpose(u_ew[0], (0, 3, 2, 1))
    uT = pad_nodes(jnp.concatenate(
        [jnp.zeros((T, H, 1, N), f32), uTr], axis=2))
    dT = pad_nodes(jnp.transpose(d_ew[0], (0, 3, 2, 1)))
    pad_idx = jnp.broadcast_to(jnp.arange(N, NPAD, dtype=jnp.int32)[None, :],
                               (K1, NPAD - N))
    nnT = jnp.concatenate([nearest_nodes.T, pad_idx], axis=1)

    a_h = alpha_x[0, :, :, 0].T
    b_h = beta_x[0, :, :, 0].T
    params = jnp.concatenate(
        [jnp.full((H, 1), mu_u[0], f32),
         jnp.full((H, 1), mu_d2[0] + rho[0] / 2.0, f32),
         a_h.astype(f32), b_h.astype(f32)], axis=1)
    params = jnp.broadcast_to(params[:, :, None], (H, 8, 16))

    mesh = plsc.VectorSubcoreMesh(core_axis_name="c", subcore_axis_name="s")
    out = pl.kernel(
        _sc_body,
        out_type=(
            jax.ShapeDtypeStruct((T, H, C, NPAD), f32),
            jax.ShapeDtypeStruct((T, H, C, NPAD), f32),
            jax.ShapeDtypeStruct((T, H, C, NPAD), f32),
            jax.ShapeDtypeStruct((T, H, C, NPAD), f32),
            jax.ShapeDtypeStruct((H, NT, C, NPAD), f32),
        ),
        mesh=mesh,
        compiler_params=pltpu.CompilerParams(use_tc_tiling_on_sc=False,
                                             needs_layout_passes=False),
        scratch_types=[
            pltpu.VMEM((C, NPAD), f32),
            pltpu.VMEM((K1, NPT), jnp.int32),
            pltpu.VMEM((K1, NPT), f32),
            pltpu.VMEM((K1, NPT), f32),
            pltpu.VMEM((C, NPT), f32),
            pltpu.VMEM((C, NPT), f32),
            pltpu.VMEM((C, NPT), f32),
            pltpu.VMEM((C, NPT), f32),
            pltpu.VMEM((C, NPT), f32),
            pltpu.VMEM((C, NPT), f32),
            pltpu.VMEM((NT, C, NPT), f32),
            pltpu.VMEM((H, 8, 16), f32),
            pltpu.SemaphoreType.DMA,
        ],
    )(xT, rhsT, uT, dT, nnT, params)

    x0 = out[0][:, :, :, :N]
    return jnp.transpose(x0, (0, 3, 1, 2))[None]
